# baseline (speedup 1.0000x reference)
import jax
import jax.numpy as jnp
from jax.experimental import pallas as pl

N = 10000
M = 160000
EL = 640000


def _spmm(rows, cols, vals, feat, n_out):
    return jnp.zeros((n_out, feat.shape[1]), feat.dtype).at[rows].add(vals[:, None] * feat[cols])


def _batchnorm(x, gamma, beta, eps=1e-5):
    mu = jnp.mean(x, axis=0)
    var = jnp.var(x, axis=0)
    return (x - mu) / jnp.sqrt(var + eps) * gamma + beta


def kernel(node_feat, node_feat_line, Fa_vals, Fb_vals, Pm_vals, Pd_vals,
           W_x2x1, W_y2x1, W_x2x2, W_y2x2, W_y2y1, W_y2y2, W_x2y1, W_x2y2,
           gamma_x, beta_x, gamma_y, beta_y,
           edge_src, edge_dst, line_src, line_dst):
    eid = jnp.arange(M)
    P_rows = jnp.concatenate([edge_src, edge_dst])
    P_cols = jnp.concatenate([eid, eid])
    x2x = jnp.concatenate([_spmm(edge_dst, edge_src, Fa_vals[k], node_feat, N) for k in range(2)], axis=1)
    y2x = jnp.concatenate([_spmm(P_rows, P_cols, Pm_vals, node_feat_line, N),
                           _spmm(P_rows, P_cols, Pd_vals, node_feat_line, N)], axis=1)
    z = jax.nn.relu(x2x @ W_x2x1 + y2x @ W_y2x1)
    zp = x2x @ W_x2x2 + y2x @ W_y2x2
    x = _batchnorm(jnp.concatenate([z, zp], axis=1), gamma_x, beta_x)
    y2y = jnp.concatenate([_spmm(line_dst, line_src, Fb_vals[k], node_feat_line, M) for k in range(2)], axis=1)
    x2y = jnp.concatenate([_spmm(P_cols, P_rows, Pm_vals, x, M),
                           _spmm(P_cols, P_rows, Pd_vals, x, M)], axis=1)
    w = jax.nn.relu(y2y @ W_y2y1 + x2y @ W_x2y1)
    wp = y2y @ W_y2y2 + x2y @ W_x2y2
    y = _batchnorm(jnp.concatenate([w, wp], axis=1), gamma_y, beta_y)
    return (x, y)


# speedup vs baseline: 4.0241x; 4.0241x over previous
import dataclasses
import functools

import jax
import jax.numpy as jnp
from jax import lax
from jax.experimental import pallas as pl
from jax.experimental.pallas import tpu as pltpu
from jax.experimental.pallas import tpu_sc as plsc

_N, _M, _EL = 10000, 160000, 640000
_NC, _NS, _L = 2, 16, 16

_f32 = jnp.float32
_i32 = jnp.int32


def _sc_params():
    cp = pltpu.CompilerParams()
    fields = pltpu.CompilerParams.__dataclass_fields__
    if "needs_layout_passes" in fields:
        cp = dataclasses.replace(cp, needs_layout_passes=False)
    if "use_tc_tiling_on_sc" in fields:
        cp = dataclasses.replace(cp, use_tc_tiling_on_sc=False)
    return cp


def _zero_vmem(ref):
    r, ccols = ref.shape

    @pl.loop(0, r)
    def _(i):
        @pl.loop(0, ccols, step=_L)
        def _(j):
            ref.at[i, pl.ds(j, _L)][...] = jnp.zeros((_L,), _f32)


def _bcast_lane(vec, t):
    idx = jnp.full((_L,), t, _i32)
    return vec.at[idx].get(mode="promise_in_bounds")


_A_B = 80
_A_EPW = _M // _NS
_A_NBLK = _A_EPW // _A_B
_A_SUP = 25


def _node_agg(feat, nfl, fa, esrc, edst):
    mesh = plsc.VectorSubcoreMesh(core_axis_name="c", subcore_axis_name="s")

    @functools.partial(
        pl.kernel,
        out_type=jax.ShapeDtypeStruct((_NC, _N, 128), _f32),
        mesh=mesh,
        scratch_types=[
            pltpu.VMEM((_A_SUP * _A_B,), _i32),
            pltpu.VMEM((_A_SUP * _A_B,), _f32),
            pltpu.VMEM((_A_SUP * _A_B,), _i32),
            pltpu.VMEM((_A_B,), _i32),
            pltpu.VMEM((_A_B, 128), _f32),
            pltpu.VMEM((_A_B, 128), _f32),
            pltpu.VMEM_SHARED((_N, 128), _f32),
            pltpu.SemaphoreType.DMA,
        ],
        compiler_params=_sc_params(),
    )
    def k(feat_h, nfl_h, fa_h, esrc_h, edst_h, out_x,
          idx_v, fa_v, dsup_v, dstb_v, rows_v, scl_v, accx, sem):
        c = lax.axis_index("c")
        s = lax.axis_index("s")
        base = s * _A_EPW

        _zero_vmem(scl_v)

        @pl.when(s < 10)
        def _():
            zrow = s * 1000

            @pl.loop(0, 25)
            def _(i):
                pltpu.sync_copy(scl_v.at[pl.ds(0, 40)],
                                accx.at[pl.ds(zrow + i * 40, 40)])

        plsc.subcore_barrier()

        @pl.loop(0, _A_NBLK)
        def _(g):
            @pl.when(g % _A_SUP == 0)
            def _():
                off = base + (g // _A_SUP) * _A_SUP * _A_B
                pltpu.sync_copy(esrc_h.at[pl.ds(off, _A_SUP * _A_B)], idx_v)
                pltpu.sync_copy(fa_h.at[pl.ds(c * _M + off, _A_SUP * _A_B)],
                                fa_v)
                pltpu.sync_copy(edst_h.at[pl.ds(off, _A_SUP * _A_B)], dsup_v)

            gg = g % _A_SUP
            @pl.loop(0, _A_B, step=_L)
            def _(j0):
                dstb_v.at[pl.ds(j0, _L)][...] = dsup_v[pl.ds(gg * _A_B + j0, _L)]

            pltpu.async_copy(feat_h.at[idx_v.at[pl.ds(gg * _A_B, _A_B)]],
                             rows_v, sem).wait()

            @pl.loop(0, _A_B, step=_L)
            def _(j0):
                cv = fa_v[pl.ds(gg * _A_B + j0, _L)]
                for t in range(_L):
                    bc = _bcast_lane(cv, t)
                    for cc in range(0, 128, _L):
                        scl_v.at[j0 + t, pl.ds(cc, _L)][...] = (
                            rows_v[j0 + t, pl.ds(cc, _L)] * bc)

            pltpu.sync_copy(scl_v, accx.at[dstb_v], add=True)

        plsc.subcore_barrier()

        @pl.when(s < 10)
        def _():
            orow = s * 1000

            @pl.loop(0, 5)
            def _(i):
                pltpu.sync_copy(accx.at[pl.ds(orow + i * 200, 200)],
                                out_x.at[c, pl.ds(orow + i * 200, 200)])

    return k(feat, nfl, fa, esrc, edst)


_SD_B = 80
_SD_NBLK = _A_EPW // _SD_B
_SD_SUP = 25


def _sd_agg(nfl, keys2, eid):
    mesh = plsc.VectorSubcoreMesh(core_axis_name="c", subcore_axis_name="s")

    @functools.partial(
        pl.kernel,
        out_type=jax.ShapeDtypeStruct((_NC, _N, 32), _f32),
        mesh=mesh,
        scratch_types=[
            pltpu.VMEM((_SD_SUP * _SD_B,), _i32),
            pltpu.VMEM((_SD_B,), _i32),
            pltpu.VMEM((_SD_B,), _i32),
            pltpu.VMEM((_SD_B, 32), _f32),
            pltpu.VMEM((40, 32), _f32),
            pltpu.VMEM_SHARED((_N, 32), _f32),
            pltpu.SemaphoreType.DMA,
        ],
        compiler_params=_sc_params(),
    )
    def k(nfl_h, keys2_h, eid_h, out_sd, ksup_v, keyb_v, idb_v, nflb_v,
          zb_v, acc, sem):
        c = lax.axis_index("c")
        s = lax.axis_index("s")
        base = s * _A_EPW
        _zero_vmem(zb_v)

        @pl.when(s < 10)
        def _():
            zrow = s * 1000

            @pl.loop(0, 25)
            def _(i):
                pltpu.sync_copy(zb_v, acc.at[pl.ds(zrow + i * 40, 40)])

        plsc.subcore_barrier()

        @pl.loop(0, _SD_NBLK)
        def _(g):
            @pl.when(g % _SD_SUP == 0)
            def _():
                off = c * _M + base + (g // _SD_SUP) * _SD_SUP * _SD_B
                pltpu.sync_copy(keys2_h.at[pl.ds(off, _SD_SUP * _SD_B)], ksup_v)

            gg = g % _SD_SUP
            @pl.loop(0, _SD_B, step=_L)
            def _(j0):
                keyb_v.at[pl.ds(j0, _L)][...] = ksup_v[pl.ds(gg * _SD_B + j0, _L)]

            pltpu.sync_copy(eid_h.at[pl.ds(base + g * _SD_B, _SD_B)], idb_v)
            pltpu.async_copy(nfl_h.at[idb_v], nflb_v, sem).wait()
            pltpu.sync_copy(nflb_v, acc.at[keyb_v], add=True)

        plsc.subcore_barrier()

        @pl.when(s < 10)
        def _():
            orow = s * 1000

            @pl.loop(0, 25)
            def _(i):
                pltpu.sync_copy(acc.at[pl.ds(orow + i * 40, 40)],
                                out_sd.at[c, pl.ds(orow + i * 40, 40)])

    return k(nfl, keys2, eid)


_B_B = 40
_B_EPW = _EL // _NS
_B_NBLK = _B_EPW // _B_B
_B_SUP = 25
_B_R = 53376
_B_G = 16


def _line_agg(nfl, fb, lsrc, ldst):
    mesh = plsc.VectorSubcoreMesh(core_axis_name="c", subcore_axis_name="s")
    passes = [(0, _B_R), (_B_R, _B_R), (2 * _B_R, _M - 2 * _B_R)]

    @functools.partial(
        pl.kernel,
        out_type=jax.ShapeDtypeStruct((_NC, _M, 32), _f32),
        mesh=mesh,
        scratch_types=[
            pltpu.VMEM((_B_SUP * _B_B,), _i32),
            pltpu.VMEM((_B_SUP * _B_B,), _i32),
            pltpu.VMEM((_B_SUP * _B_B,), _f32),
            pltpu.VMEM((_B_B,), _i32),
            pltpu.VMEM((_B_B, 32), _f32),
            pltpu.VMEM((_B_B, 32), _f32),
            pltpu.VMEM((40, 32), _f32),
            pltpu.VMEM_SHARED((_B_R + _B_G, 32), _f32),
            pltpu.SemaphoreType.DMA,
        ],
        compiler_params=_sc_params(),
    )
    def k(nfl_h, fb_h, lsrc_h, ldst_h, out_y, src_v, dst_v, fb_v, loc_v,
          rows_v, scl_v, zb_v, acc, sem):
        c = lax.axis_index("c")
        s = lax.axis_index("s")
        base = s * _B_EPW
        _zero_vmem(zb_v)
        lane = lax.broadcasted_iota(_i32, (_L,), 0)

        for p, (lo, nrows) in enumerate(passes):
            zrow = s * 3336

            @pl.loop(0, 83)
            def _(i):
                pltpu.sync_copy(zb_v, acc.at[pl.ds(zrow + i * 40, 40)])

            pltpu.sync_copy(zb_v.at[pl.ds(0, 16)],
                            acc.at[pl.ds(zrow + 3320, 16)])

            @pl.when(s == _NS - 1)
            def _():
                pltpu.sync_copy(zb_v.at[pl.ds(0, _B_G)],
                                acc.at[pl.ds(_B_R, _B_G)])

            plsc.subcore_barrier()

            @pl.loop(0, _B_NBLK)
            def _(g):
                @pl.when(g % _B_SUP == 0)
                def _():
                    off = base + (g // _B_SUP) * _B_SUP * _B_B
                    pltpu.sync_copy(lsrc_h.at[pl.ds(off, _B_SUP * _B_B)], src_v)
                    pltpu.sync_copy(ldst_h.at[pl.ds(off, _B_SUP * _B_B)], dst_v)
                    pltpu.sync_copy(fb_h.at[pl.ds(c * _EL + off, _B_SUP * _B_B)],
                                    fb_v)

                gg = g % _B_SUP
                @pl.loop(0, _B_B, step=_L)
                def _(j0):
                    d = dst_v[pl.ds(gg * _B_B + j0, _L)]
                    local = d - lo
                    owned = (local >= 0) & (local < _B_R)
                    gidx = _B_R + (lane & (_B_G - 1))
                    loc_v.at[pl.ds(j0, _L)][...] = jnp.where(owned, local, gidx)

                pltpu.async_copy(nfl_h.at[src_v.at[pl.ds(gg * _B_B, _B_B)]],
                                 rows_v, sem).wait()

                @pl.loop(0, _B_B, step=_L)
                def _(j0):
                    cv = fb_v[pl.ds(gg * _B_B + j0, _L)]
                    for t in range(_L):
                        bc = _bcast_lane(cv, t)
                        for cc in range(0, 32, _L):
                            scl_v.at[j0 + t, pl.ds(cc, _L)][...] = (
                                rows_v[j0 + t, pl.ds(cc, _L)] * bc)

                pltpu.sync_copy(scl_v, acc.at[loc_v], add=True)

            plsc.subcore_barrier()

            cpw = nrows // _NS
            orow = s * cpw
            nco = cpw // 40
            rem = cpw - nco * 40

            @pl.loop(0, nco)
            def _(i):
                pltpu.sync_copy(acc.at[pl.ds(orow + i * 40, 40)],
                                out_y.at[c, pl.ds(lo + orow + i * 40, 40)])

            if rem:
                pltpu.sync_copy(acc.at[pl.ds(orow + nco * 40, rem)],
                                out_y.at[c, pl.ds(lo + orow + nco * 40, rem)])

            if p + 1 < len(passes):
                plsc.subcore_barrier()

    return k(nfl, fb, lsrc, ldst)


_C_B = 80
_C_RPW = 2 * _M // (_NC * _NS)
_C_NBLK = _C_RPW // _C_B


def _gather_uv(x, esrc, edst):
    mesh = plsc.VectorSubcoreMesh(core_axis_name="c", subcore_axis_name="s")

    @functools.partial(
        pl.kernel,
        out_type=[
            jax.ShapeDtypeStruct((_M, 128), _f32),
            jax.ShapeDtypeStruct((_M, 128), _f32),
        ],
        mesh=mesh,
        scratch_types=[
            pltpu.VMEM((_C_RPW,), _i32),
            pltpu.VMEM((_C_B, 128), _f32),
            pltpu.SemaphoreType.DMA,
        ],
        compiler_params=_sc_params(),
    )
    def k(x_h, esrc_h, edst_h, out_u, out_v, idx_v, rows_v, sem):
        c = lax.axis_index("c")
        s = lax.axis_index("s")
        w = s * _NC + c

        @pl.when(w < _NS)
        def _():
            base = w * _C_RPW
            pltpu.sync_copy(esrc_h.at[pl.ds(base, _C_RPW)], idx_v)

            @pl.loop(0, _C_NBLK)
            def _(g):
                pltpu.async_copy(x_h.at[idx_v.at[pl.ds(g * _C_B, _C_B)]],
                                 rows_v, sem).wait()
                pltpu.sync_copy(rows_v, out_u.at[pl.ds(base + g * _C_B, _C_B)])

        @pl.when(w >= _NS)
        def _():
            base = (w - _NS) * _C_RPW
            pltpu.sync_copy(edst_h.at[pl.ds(base, _C_RPW)], idx_v)

            @pl.loop(0, _C_NBLK)
            def _(g):
                pltpu.async_copy(x_h.at[idx_v.at[pl.ds(g * _C_B, _C_B)]],
                                 rows_v, sem).wait()
                pltpu.sync_copy(rows_v, out_v.at[pl.ds(base + g * _C_B, _C_B)])

    return k(x, esrc, edst)


_HI = jax.lax.Precision.HIGHEST
_EPS = 1e-5


def _dot(a, b):
    return jax.lax.dot(a, b, precision=_HI, preferred_element_type=_f32)


_ND_BM = 2000


def _node_phase1_kernel(x2x_ref, sd_ref, wx1_ref, wy1_ref, wx2_ref, wy2_ref,
                        cat_ref, stats_ref):
    i = pl.program_id(0)
    x0 = x2x_ref[0]
    x1 = x2x_ref[1]
    S = sd_ref[0]
    D = sd_ref[1]
    wx1 = wx1_ref[...]
    wx2 = wx2_ref[...]
    wy1 = wy1_ref[...]
    wy2 = wy2_ref[...]
    z = (_dot(x0, wx1[:128]) + _dot(x1, wx1[128:])
         + _dot(S, wy1[:32] + wy1[32:]) + _dot(D, wy1[:32] - wy1[32:]))
    zp = (_dot(x0, wx2[:128]) + _dot(x1, wx2[128:])
          + _dot(S, wy2[:32] + wy2[32:]) + _dot(D, wy2[:32] - wy2[32:]))
    cat = jnp.concatenate([jax.nn.relu(z), zp], axis=1)
    cat_ref[...] = cat

    @pl.when(i == 0)
    def _():
        stats_ref[...] = jnp.zeros_like(stats_ref)

    stats_ref[...] += jnp.stack([jnp.sum(cat, axis=0),
                                 jnp.sum(cat * cat, axis=0)])


def _node_phase2_kernel(cat_ref, stats_ref, gamma_ref, beta_ref, out_ref):
    mu = stats_ref[0] / _N
    var = stats_ref[1] / _N - mu * mu
    out_ref[...] = ((cat_ref[...] - mu[None]) / jnp.sqrt(var[None] + _EPS)
                    * gamma_ref[...] + beta_ref[...])


def _node_dense(x2x, sd, wx1, wy1, wx2, wy2, gamma, beta):
    nb = _N // _ND_BM
    cat, stats = pl.pallas_call(
        _node_phase1_kernel,
        grid=(nb,),
        in_specs=[
            pl.BlockSpec((_NC, _ND_BM, 128), lambda i: (0, i, 0)),
            pl.BlockSpec((_NC, _ND_BM, 32), lambda i: (0, i, 0)),
            pl.BlockSpec(wx1.shape, lambda i: (0, 0)),
            pl.BlockSpec(wy1.shape, lambda i: (0, 0)),
            pl.BlockSpec(wx2.shape, lambda i: (0, 0)),
            pl.BlockSpec(wy2.shape, lambda i: (0, 0)),
        ],
        out_specs=[
            pl.BlockSpec((_ND_BM, 128), lambda i: (i, 0)),
            pl.BlockSpec((2, 128), lambda i: (0, 0)),
        ],
        out_shape=[
            jax.ShapeDtypeStruct((_N, 128), _f32),
            jax.ShapeDtypeStruct((2, 128), _f32),
        ],
    )(x2x, sd, wx1, wy1, wx2, wy2)
    return pl.pallas_call(
        _node_phase2_kernel,
        grid=(nb,),
        in_specs=[
            pl.BlockSpec((_ND_BM, 128), lambda i: (i, 0)),
            pl.BlockSpec((2, 128), lambda i: (0, 0)),
            pl.BlockSpec((1, 128), lambda i: (0, 0)),
            pl.BlockSpec((1, 128), lambda i: (0, 0)),
        ],
        out_specs=pl.BlockSpec((_ND_BM, 128), lambda i: (i, 0)),
        out_shape=jax.ShapeDtypeStruct((_N, 128), _f32),
    )(cat, stats, gamma.reshape(1, -1), beta.reshape(1, -1))


_L1_BM = 2000


def _line_phase1_kernel(y2y_ref, u_ref, v_ref, wy1_ref, wx1_ref, wy2_ref,
                        wx2_ref, wcat_ref, stats_ref):
    i = pl.program_id(0)
    y0 = y2y_ref[0]
    y1 = y2y_ref[1]
    U = u_ref[...]
    V = v_ref[...]
    wy1 = wy1_ref[...]
    wx1 = wx1_ref[...]
    wy2 = wy2_ref[...]
    wx2 = wx2_ref[...]
    w = (_dot(y0, wy1[:32]) + _dot(y1, wy1[32:])
         + _dot(U, wx1[:128] + wx1[128:]) + _dot(V, wx1[:128] - wx1[128:]))
    wp = (_dot(y0, wy2[:32]) + _dot(y1, wy2[32:])
          + _dot(U, wx2[:128] + wx2[128:]) + _dot(V, wx2[:128] - wx2[128:]))
    cat = jnp.concatenate([jax.nn.relu(w), wp], axis=1)
    wcat_ref[...] = cat

    @pl.when(i == 0)
    def _():
        stats_ref[...] = jnp.zeros_like(stats_ref)

    part = jnp.stack([jnp.sum(cat, axis=0), jnp.sum(cat * cat, axis=0)])
    stats_ref[...] += part


def _line_phase2_kernel(wcat_ref, stats_ref, gamma_ref, beta_ref, out_ref):
    mu = stats_ref[0] / _M
    var = stats_ref[1] / _M - mu * mu
    out_ref[...] = ((wcat_ref[...] - mu[None]) / jnp.sqrt(var[None] + _EPS)
                    * gamma_ref[...] + beta_ref[...])


def _line_dense(y2y, u, v, wy1, wx1, wy2, wx2, gamma, beta):
    nb = _M // _L1_BM
    wcat, stats = pl.pallas_call(
        _line_phase1_kernel,
        grid=(nb,),
        in_specs=[
            pl.BlockSpec((_NC, _L1_BM, 32), lambda i: (0, i, 0)),
            pl.BlockSpec((_L1_BM, 128), lambda i: (i, 0)),
            pl.BlockSpec((_L1_BM, 128), lambda i: (i, 0)),
            pl.BlockSpec(wy1.shape, lambda i: (0, 0)),
            pl.BlockSpec(wx1.shape, lambda i: (0, 0)),
            pl.BlockSpec(wy2.shape, lambda i: (0, 0)),
            pl.BlockSpec(wx2.shape, lambda i: (0, 0)),
        ],
        out_specs=[
            pl.BlockSpec((_L1_BM, 64), lambda i: (i, 0)),
            pl.BlockSpec((2, 64), lambda i: (0, 0)),
        ],
        out_shape=[
            jax.ShapeDtypeStruct((_M, 64), _f32),
            jax.ShapeDtypeStruct((2, 64), _f32),
        ],
    )(y2y, u, v, wy1, wx1, wy2, wx2)
    return pl.pallas_call(
        _line_phase2_kernel,
        grid=(nb,),
        in_specs=[
            pl.BlockSpec((_L1_BM, 64), lambda i: (i, 0)),
            pl.BlockSpec((2, 64), lambda i: (0, 0)),
            pl.BlockSpec((1, 64), lambda i: (0, 0)),
            pl.BlockSpec((1, 64), lambda i: (0, 0)),
        ],
        out_specs=pl.BlockSpec((_L1_BM, 64), lambda i: (i, 0)),
        out_shape=jax.ShapeDtypeStruct((_M, 64), _f32),
    )(wcat, stats, gamma.reshape(1, -1), beta.reshape(1, -1))


def kernel(node_feat, node_feat_line, Fa_vals, Fb_vals, Pm_vals, Pd_vals,
           W_x2x1, W_y2x1, W_x2x2, W_y2x2, W_y2y1, W_y2y2, W_x2y1, W_x2y2,
           gamma_x, beta_x, gamma_y, beta_y,
           edge_src, edge_dst, line_src, line_dst):
    esrc = edge_src.astype(_i32)
    edst = edge_dst.astype(_i32)
    lsrc = line_src.astype(_i32)
    ldst = line_dst.astype(_i32)

    x2x = _node_agg(node_feat, node_feat_line, Fa_vals.reshape(-1), esrc, edst)
    sd = _sd_agg(node_feat_line, jnp.concatenate([esrc, edst]),
                 jnp.arange(_M, dtype=_i32))
    y2y = _line_agg(node_feat_line, Fb_vals.reshape(-1), lsrc, ldst)
    x = _node_dense(x2x, sd, W_x2x1, W_y2x1, W_x2x2, W_y2x2, gamma_x, beta_x)
    u, v = _gather_uv(x, esrc, edst)
    y = _line_dense(y2y, u, v, W_y2y1, W_x2y1, W_y2y2, W_x2y2, gamma_y, beta_y)
    return (x, y)
